# baseline (speedup 1.0000x reference)
import functools

import jax
import jax.numpy as jnp
from jax import lax
from jax.experimental import pallas as pl
from jax.experimental.pallas import tpu as pltpu
from jax.experimental.pallas import tpu_sc as plsc

N = 10000
E = 160000
F_IN = 256
H1 = 64
H2 = 32

NC = 2
NS = 16
NW = NC * NS

NPAD = 10112
RPS = NPAD // NS
K = 128
EW = 5120
CH = EW // K
EPAD = NW * EW

_MESH = dict(core_axis_name="c", subcore_axis_name="s")


DW = 16

@functools.partial(
    pl.kernel,
    out_type=jax.ShapeDtypeStruct((NC, NPAD, DW), jnp.float32),
    mesh=plsc.VectorSubcoreMesh(**_MESH),
    compiler_params=pltpu.CompilerParams(use_tc_tiling_on_sc=False),
    scratch_types=[
        pltpu.VMEM((CH, K), jnp.int32),
        pltpu.VMEM((K, DW), jnp.float32),
        pltpu.VMEM((RPS, DW), jnp.float32),
        pltpu.VMEM_SHARED((NPAD, DW), jnp.float32),
        pltpu.SemaphoreType.DMA,
    ],
)
def _deg_kernel(dst_hbm, out_hbm, dst_v, ones_v, zbuf_v, acc_sh, ssem):
    c = lax.axis_index("c")
    s = lax.axis_index("s")
    wid = c * NS + s
    row0 = s * RPS
    one16 = jnp.ones((16,), jnp.float32)
    zero16 = jnp.zeros((16,), jnp.float32)

    def fill(i, carry):
        ones_v[i] = one16
        return carry

    lax.fori_loop(0, K, fill, 0)

    def zfill(i, carry):
        zbuf_v[i] = zero16
        return carry

    lax.fori_loop(0, RPS, zfill, 0)
    pltpu.sync_copy(zbuf_v, acc_sh.at[pl.ds(row0, RPS)])
    pltpu.sync_copy(dst_hbm.at[wid], dst_v)
    plsc.subcore_barrier()

    def chunk(j, carry):
        pltpu.async_copy(ones_v, acc_sh.at[dst_v.at[j]], ssem, add=True)
        return carry

    lax.fori_loop(0, CH, chunk, 0)

    def drain(j, carry):
        pltpu.make_async_copy(ones_v, acc_sh.at[pl.ds(0, K)], ssem).wait()
        return carry

    lax.fori_loop(0, CH, drain, 0)
    plsc.subcore_barrier()
    pltpu.sync_copy(acc_sh.at[pl.ds(row0, RPS)], out_hbm.at[c, pl.ds(row0, RPS)])


def _make_prop(F):
    @functools.partial(
        pl.kernel,
        out_type=jax.ShapeDtypeStruct((NC, NPAD, F), jnp.float32),
        mesh=plsc.VectorSubcoreMesh(**_MESH),
        compiler_params=pltpu.CompilerParams(use_tc_tiling_on_sc=False),
        scratch_types=[
            pltpu.VMEM((CH, K), jnp.int32),
            pltpu.VMEM((CH, K), jnp.int32),
            pltpu.VMEM((K, F), jnp.float32),
            pltpu.VMEM((K, F), jnp.float32),
            pltpu.VMEM_SHARED((NPAD, F), jnp.float32),
            pltpu.SemaphoreType.DMA,
            pltpu.SemaphoreType.DMA,
        ],
    )
    def _prop(h_hbm, src_hbm, dst_hbm, out_hbm, src_v, dst_v, rows0_v, rows1_v,
              acc_sh, gsem0, gsem1):
        c = lax.axis_index("c")
        s = lax.axis_index("s")
        wid = c * NS + s
        row0 = s * RPS
        pltpu.sync_copy(h_hbm.at[pl.ds(row0, RPS)], acc_sh.at[pl.ds(row0, RPS)])
        pltpu.sync_copy(src_hbm.at[wid], src_v)
        pltpu.sync_copy(dst_hbm.at[wid], dst_v)
        plsc.subcore_barrier()

        pltpu.async_copy(h_hbm.at[src_v.at[0]], rows0_v, gsem0)

        def pair(t, carry):
            j0 = t * 2
            pltpu.async_copy(h_hbm.at[src_v.at[j0 + 1]], rows1_v, gsem1)
            pltpu.make_async_copy(h_hbm.at[pl.ds(0, K)], rows0_v, gsem0).wait()
            pltpu.sync_copy(rows0_v, acc_sh.at[dst_v.at[j0]], add=True)

            @pl.when(t + 1 < CH // 2)
            def _():
                pltpu.async_copy(h_hbm.at[src_v.at[j0 + 2]], rows0_v, gsem0)

            pltpu.make_async_copy(h_hbm.at[pl.ds(0, K)], rows1_v, gsem1).wait()
            pltpu.sync_copy(rows1_v, acc_sh.at[dst_v.at[j0 + 1]], add=True)
            return carry

        lax.fori_loop(0, CH // 2, pair, 0)
        plsc.subcore_barrier()
        pltpu.sync_copy(acc_sh.at[pl.ds(row0, RPS)], out_hbm.at[c, pl.ds(row0, RPS)])

    return _prop


_prop64 = _make_prop(H1)
_prop32 = _make_prop(H2)


def _b_body(hist_ref, x_ref, w1_ref, h1_ref, dinv_ref):
    deg = hist_ref[0, :, 0:1] + hist_ref[1, :, 0:1] + 1.0
    di = lax.rsqrt(jnp.maximum(deg, 1e-12))
    h = jnp.dot(x_ref[...], w1_ref[...], preferred_element_type=jnp.float32)
    h1_ref[...] = h * di
    dinv_ref[...] = di


_b_call = pl.pallas_call(
    _b_body,
    out_shape=[
        jax.ShapeDtypeStruct((NPAD, H1), jnp.float32),
        jax.ShapeDtypeStruct((NPAD, 1), jnp.float32),
    ],
)


def _d_body(p_ref, h1_ref, dinv_ref, b1_ref, w2_ref, h2_ref):
    di = dinv_ref[...]
    ssum = p_ref[0] + p_ref[1] - h1_ref[...]
    a1 = jnp.maximum(ssum * di + b1_ref[...], 0.0)
    h2_ref[...] = jnp.dot(a1, w2_ref[...], preferred_element_type=jnp.float32) * di


_d_call = pl.pallas_call(
    _d_body,
    out_shape=jax.ShapeDtypeStruct((NPAD, H2), jnp.float32),
)


def _f_body(q_ref, h2_ref, dinv_ref, b2_ref, wfc_ref, bfc_ref, out_ref):
    di = dinv_ref[...]
    ssum = q_ref[0] + q_ref[1] - h2_ref[...]
    a2 = jnp.maximum(ssum * di + b2_ref[...], 0.0)
    out_ref[...] = jnp.dot(a2, wfc_ref[...], preferred_element_type=jnp.float32) + bfc_ref[...]


_f_call = pl.pallas_call(
    _f_body,
    out_shape=jax.ShapeDtypeStruct((NPAD, 1), jnp.float32),
)


def kernel(x, edge_index, W1, b1, W2, b2, Wfc, bfc):
    src = edge_index[0]
    dst = edge_index[1]
    pad = jnp.full((EPAD - E,), N, dtype=jnp.int32)
    srcp = jnp.concatenate([src, pad]).reshape(NW, CH, K)
    dstp = jnp.concatenate([dst, pad]).reshape(NW, CH, K)
    xp = jnp.zeros((NPAD, F_IN), x.dtype).at[:N].set(x)

    hist = _deg_kernel(dstp)
    h1p, dinv = _b_call(hist, xp, W1)
    p = _prop64(h1p, srcp, dstp)
    h2p = _d_call(p, h1p, dinv, b1.reshape(1, H1), W2)
    q = _prop32(h2p, srcp, dstp)
    outp = _f_call(q, h2p, dinv, b2.reshape(1, H2), Wfc, bfc.reshape(1, 1))
    return outp[:N, 0]


# speedup vs baseline: 1.6375x; 1.6375x over previous
import functools

import jax
import jax.numpy as jnp
from jax import lax
from jax.experimental import pallas as pl
from jax.experimental.pallas import tpu as pltpu
from jax.experimental.pallas import tpu_sc as plsc

N = 10000
E = 160000
F_IN = 256
H1 = 64
H2 = 32

NC = 2
NS = 16
NW = NC * NS

NPAD = 10112
RPS = NPAD // NS
K = 128
EW = 5120
CH = EW // K
EPAD = NW * EW

_MESH = dict(core_axis_name="c", subcore_axis_name="s")


DW = 16

@functools.partial(
    pl.kernel,
    out_type=jax.ShapeDtypeStruct((NC, NPAD, DW), jnp.float32),
    mesh=plsc.VectorSubcoreMesh(**_MESH),
    compiler_params=pltpu.CompilerParams(use_tc_tiling_on_sc=False),
    scratch_types=[
        pltpu.VMEM((CH, K), jnp.int32),
        pltpu.VMEM((K, DW), jnp.float32),
        pltpu.VMEM((RPS, DW), jnp.float32),
        pltpu.VMEM_SHARED((NPAD, DW), jnp.float32),
        pltpu.SemaphoreType.DMA,
    ],
)
def _deg_kernel(dst_hbm, out_hbm, dst_v, ones_v, zbuf_v, acc_sh, ssem):
    c = lax.axis_index("c")
    s = lax.axis_index("s")
    wid = c * NS + s
    row0 = s * RPS
    one16 = jnp.ones((16,), jnp.float32)
    zero16 = jnp.zeros((16,), jnp.float32)

    def fill(i, carry):
        ones_v[i] = one16
        return carry

    lax.fori_loop(0, K, fill, 0)

    def zfill(i, carry):
        zbuf_v[i] = zero16
        return carry

    lax.fori_loop(0, RPS, zfill, 0)
    pltpu.sync_copy(zbuf_v, acc_sh.at[pl.ds(row0, RPS)])
    pltpu.sync_copy(dst_hbm.at[wid], dst_v)
    plsc.subcore_barrier()

    def chunk(j, carry):
        pltpu.async_copy(ones_v, acc_sh.at[dst_v.at[j]], ssem, add=True)
        return carry

    lax.fori_loop(0, CH, chunk, 0)

    def drain(j, carry):
        pltpu.make_async_copy(ones_v, acc_sh.at[pl.ds(0, K)], ssem).wait()
        return carry

    lax.fori_loop(0, CH, drain, 0)
    plsc.subcore_barrier()
    pltpu.sync_copy(acc_sh.at[pl.ds(row0, RPS)], out_hbm.at[c, pl.ds(row0, RPS)])


def _make_prop(F):
    @functools.partial(
        pl.kernel,
        out_type=jax.ShapeDtypeStruct((NC, NPAD, F), jnp.float32),
        mesh=plsc.VectorSubcoreMesh(**_MESH),
        compiler_params=pltpu.CompilerParams(use_tc_tiling_on_sc=False),
        scratch_types=[
            pltpu.VMEM((CH, K), jnp.int32),
            pltpu.VMEM((CH, K), jnp.int32),
            pltpu.VMEM((K, F), jnp.float32),
            pltpu.VMEM((K, F), jnp.float32),
            pltpu.VMEM_SHARED((NPAD, F), jnp.float32),
            pltpu.SemaphoreType.DMA,
            pltpu.SemaphoreType.DMA,
        ],
    )
    def _prop(h_hbm, src_hbm, dst_hbm, out_hbm, src_v, dst_v, rows0_v, rows1_v,
              acc_sh, gsem0, gsem1):
        c = lax.axis_index("c")
        s = lax.axis_index("s")
        wid = c * NS + s
        row0 = s * RPS
        pltpu.sync_copy(h_hbm.at[pl.ds(row0, RPS)], acc_sh.at[pl.ds(row0, RPS)])
        pltpu.sync_copy(src_hbm.at[wid], src_v)
        pltpu.sync_copy(dst_hbm.at[wid], dst_v)
        plsc.subcore_barrier()

        pltpu.async_copy(h_hbm.at[src_v.at[0]], rows0_v, gsem0)

        def pair(t, carry):
            j0 = t * 2
            pltpu.async_copy(h_hbm.at[src_v.at[j0 + 1]], rows1_v, gsem1)
            pltpu.make_async_copy(h_hbm.at[pl.ds(0, K)], rows0_v, gsem0).wait()
            pltpu.sync_copy(rows0_v, acc_sh.at[dst_v.at[j0]], add=True)

            @pl.when(t + 1 < CH // 2)
            def _():
                pltpu.async_copy(h_hbm.at[src_v.at[j0 + 2]], rows0_v, gsem0)

            pltpu.make_async_copy(h_hbm.at[pl.ds(0, K)], rows1_v, gsem1).wait()
            pltpu.sync_copy(rows1_v, acc_sh.at[dst_v.at[j0 + 1]], add=True)
            return carry

        lax.fori_loop(0, CH // 2, pair, 0)
        plsc.subcore_barrier()
        pltpu.sync_copy(acc_sh.at[pl.ds(row0, RPS)], out_hbm.at[c, pl.ds(row0, RPS)])

    return _prop


_prop64 = _make_prop(H1)
_prop32 = _make_prop(H2)


def _b_body(hist_ref, x_ref, w1_ref, h1_ref, dinv_ref):
    deg = hist_ref[0, :, 0:1] + hist_ref[1, :, 0:1] + 1.0
    di = lax.rsqrt(jnp.maximum(deg, 1e-12))
    h = jnp.dot(x_ref[...], w1_ref[...], preferred_element_type=jnp.float32)
    h1_ref[...] = h * di
    dinv_ref[...] = di


_b_call = pl.pallas_call(
    _b_body,
    out_shape=[
        jax.ShapeDtypeStruct((NPAD, H1), jnp.float32),
        jax.ShapeDtypeStruct((NPAD, 1), jnp.float32),
    ],
)


def _d_body(p_ref, h1_ref, dinv_ref, b1_ref, w2_ref, h2_ref):
    di = dinv_ref[...]
    ssum = p_ref[0] + p_ref[1] - h1_ref[...]
    a1 = jnp.maximum(ssum * di + b1_ref[...], 0.0)
    h2_ref[...] = jnp.dot(a1, w2_ref[...], preferred_element_type=jnp.float32) * di


_d_call = pl.pallas_call(
    _d_body,
    out_shape=jax.ShapeDtypeStruct((NPAD, H2), jnp.float32),
)


def _f_body(q_ref, h2_ref, dinv_ref, b2_ref, wfc_ref, bfc_ref, out_ref):
    di = dinv_ref[...]
    ssum = q_ref[0] + q_ref[1] - h2_ref[...]
    a2 = jnp.maximum(ssum * di + b2_ref[...], 0.0)
    out_ref[...] = jnp.dot(a2, wfc_ref[...], preferred_element_type=jnp.float32) + bfc_ref[...]


_f_call = pl.pallas_call(
    _f_body,
    out_shape=jax.ShapeDtypeStruct((NPAD, 1), jnp.float32),
)


def kernel(x, edge_index, W1, b1, W2, b2, Wfc, bfc):
    src = edge_index[0]
    dst = edge_index[1]
    pad = N + jnp.arange(EPAD - E, dtype=jnp.int32) % (NPAD - N)
    srcp = jnp.concatenate([src, pad]).reshape(NW, CH, K)
    dstp = jnp.concatenate([dst, pad]).reshape(NW, CH, K)
    xp = jnp.zeros((NPAD, F_IN), x.dtype).at[:N].set(x)

    hist = _deg_kernel(dstp)
    h1p, dinv = _b_call(hist, xp, W1)
    p = _prop64(h1p, srcp, dstp)
    h2p = _d_call(p, h1p, dinv, b1.reshape(1, H1), W2)
    q = _prop32(h2p, srcp, dstp)
    outp = _f_call(q, h2p, dinv, b2.reshape(1, H2), Wfc, bfc.reshape(1, 1))
    return outp[:N, 0]


# speedup vs baseline: 1.6718x; 1.0210x over previous
import functools

import jax
import jax.numpy as jnp
from jax import lax
from jax.experimental import pallas as pl
from jax.experimental.pallas import tpu as pltpu
from jax.experimental.pallas import tpu_sc as plsc

N = 10000
E = 160000
F_IN = 256
H1 = 64
H2 = 32

NC = 2
NS = 16
NW = NC * NS

NPAD = 10112
RPS = NPAD // NS
K = 128
EW = 5120
CH = EW // K
EPAD = NW * EW

_MESH = dict(core_axis_name="c", subcore_axis_name="s")


DW = 16

@functools.partial(
    pl.kernel,
    out_type=jax.ShapeDtypeStruct((NC, NPAD, DW), jnp.float32),
    mesh=plsc.VectorSubcoreMesh(**_MESH),
    compiler_params=pltpu.CompilerParams(use_tc_tiling_on_sc=False),
    scratch_types=[
        pltpu.VMEM((CH, K), jnp.int32),
        pltpu.VMEM((K, DW), jnp.float32),
        pltpu.VMEM((RPS, DW), jnp.float32),
        pltpu.VMEM_SHARED((NPAD, DW), jnp.float32),
        pltpu.SemaphoreType.DMA,
    ],
)
def _deg_kernel(dst_hbm, out_hbm, dst_v, ones_v, zbuf_v, acc_sh, ssem):
    c = lax.axis_index("c")
    s = lax.axis_index("s")
    wid = c * NS + s
    row0 = s * RPS
    one16 = jnp.ones((16,), jnp.float32)
    zero16 = jnp.zeros((16,), jnp.float32)

    def fill(i, carry):
        ones_v[i] = one16
        return carry

    lax.fori_loop(0, K, fill, 0)

    def zfill(i, carry):
        zbuf_v[i] = zero16
        return carry

    lax.fori_loop(0, RPS, zfill, 0)
    pltpu.sync_copy(zbuf_v, acc_sh.at[pl.ds(row0, RPS)])
    pltpu.sync_copy(dst_hbm.at[wid], dst_v)
    plsc.subcore_barrier()

    def chunk(j, carry):
        pltpu.async_copy(ones_v, acc_sh.at[dst_v.at[j]], ssem, add=True)
        return carry

    lax.fori_loop(0, CH, chunk, 0)

    def drain(j, carry):
        pltpu.make_async_copy(ones_v, acc_sh.at[pl.ds(0, K)], ssem).wait()
        return carry

    lax.fori_loop(0, CH, drain, 0)
    plsc.subcore_barrier()
    pltpu.sync_copy(acc_sh.at[pl.ds(row0, RPS)], out_hbm.at[c, pl.ds(row0, RPS)])


def _make_prop(F):
    @functools.partial(
        pl.kernel,
        out_type=jax.ShapeDtypeStruct((NC, NPAD, F), jnp.float32),
        mesh=plsc.VectorSubcoreMesh(**_MESH),
        compiler_params=pltpu.CompilerParams(use_tc_tiling_on_sc=False),
        scratch_types=[
            pltpu.VMEM((CH, K), jnp.int32),
            pltpu.VMEM((CH, K), jnp.int32),
            pltpu.VMEM((K, F), jnp.float32),
            pltpu.VMEM((K, F), jnp.float32),
            pltpu.VMEM_SHARED((NPAD, F), jnp.float32),
            pltpu.SemaphoreType.DMA,
            pltpu.SemaphoreType.DMA,
        ],
    )
    def _prop(h_hbm, src_hbm, dst_hbm, out_hbm, src_v, dst_v, rows0_v, rows1_v,
              acc_sh, gsem0, gsem1):
        c = lax.axis_index("c")
        s = lax.axis_index("s")
        wid = c * NS + s
        row0 = s * RPS
        pltpu.sync_copy(h_hbm.at[pl.ds(row0, RPS)], acc_sh.at[pl.ds(row0, RPS)])
        pltpu.sync_copy(src_hbm.at[wid], src_v)
        pltpu.sync_copy(dst_hbm.at[wid], dst_v)
        plsc.subcore_barrier()

        pltpu.async_copy(h_hbm.at[src_v.at[0]], rows0_v, gsem0)

        def pair(t, carry):
            j0 = t * 2
            pltpu.async_copy(h_hbm.at[src_v.at[j0 + 1]], rows1_v, gsem1)
            pltpu.make_async_copy(h_hbm.at[pl.ds(0, K)], rows0_v, gsem0).wait()
            pltpu.sync_copy(rows0_v, acc_sh.at[dst_v.at[j0]], add=True)

            @pl.when(t + 1 < CH // 2)
            def _():
                pltpu.async_copy(h_hbm.at[src_v.at[j0 + 2]], rows0_v, gsem0)

            pltpu.make_async_copy(h_hbm.at[pl.ds(0, K)], rows1_v, gsem1).wait()
            pltpu.sync_copy(rows1_v, acc_sh.at[dst_v.at[j0 + 1]], add=True)
            return carry

        lax.fori_loop(0, CH // 2, pair, 0)
        plsc.subcore_barrier()
        pltpu.sync_copy(acc_sh.at[pl.ds(row0, RPS)], out_hbm.at[c, pl.ds(row0, RPS)])

    return _prop


_prop64 = _make_prop(H1)
_prop32 = _make_prop(H2)


def _b1_body(x_ref, w1_ref, xw_ref):
    h = jnp.dot(x_ref[...], w1_ref[...], preferred_element_type=jnp.float32)
    xw_ref[0:N, :] = h
    xw_ref[N:NPAD, :] = jnp.zeros((NPAD - N, H1), jnp.float32)


_b1_call = pl.pallas_call(
    _b1_body,
    out_shape=jax.ShapeDtypeStruct((NPAD, H1), jnp.float32),
)


def _b2_body(hist_ref, xw_ref, h1_ref, dinv_ref):
    deg = hist_ref[0, :, 0:1] + hist_ref[1, :, 0:1] + 1.0
    di = lax.rsqrt(jnp.maximum(deg, 1e-12))
    h1_ref[...] = xw_ref[...] * di
    dinv_ref[...] = di


_b2_call = pl.pallas_call(
    _b2_body,
    out_shape=[
        jax.ShapeDtypeStruct((NPAD, H1), jnp.float32),
        jax.ShapeDtypeStruct((NPAD, 1), jnp.float32),
    ],
)


def _d_body(p_ref, h1_ref, dinv_ref, b1_ref, w2_ref, h2_ref):
    di = dinv_ref[...]
    ssum = p_ref[0] + p_ref[1] - h1_ref[...]
    a1 = jnp.maximum(ssum * di + b1_ref[...], 0.0)
    h2_ref[...] = jnp.dot(a1, w2_ref[...], preferred_element_type=jnp.float32) * di


_d_call = pl.pallas_call(
    _d_body,
    out_shape=jax.ShapeDtypeStruct((NPAD, H2), jnp.float32),
)


def _f_body(q_ref, h2_ref, dinv_ref, b2_ref, wfc_ref, bfc_ref, out_ref):
    di = dinv_ref[...]
    ssum = q_ref[0] + q_ref[1] - h2_ref[...]
    a2 = jnp.maximum(ssum * di + b2_ref[...], 0.0)
    out_ref[...] = jnp.dot(a2, wfc_ref[...], preferred_element_type=jnp.float32) + bfc_ref[...]


_f_call = pl.pallas_call(
    _f_body,
    out_shape=jax.ShapeDtypeStruct((NPAD, 1), jnp.float32),
)


def kernel(x, edge_index, W1, b1, W2, b2, Wfc, bfc):
    src = edge_index[0]
    dst = edge_index[1]
    pad = N + jnp.arange(EPAD - E, dtype=jnp.int32) % (NPAD - N)
    srcp = jnp.concatenate([src, pad]).reshape(NW, CH, K)
    dstp = jnp.concatenate([dst, pad]).reshape(NW, CH, K)

    hist = _deg_kernel(dstp)
    xw = _b1_call(x, W1)
    h1p, dinv = _b2_call(hist, xw)
    p = _prop64(h1p, srcp, dstp)
    h2p = _d_call(p, h1p, dinv, b1.reshape(1, H1), W2)
    q = _prop32(h2p, srcp, dstp)
    outp = _f_call(q, h2p, dinv, b2.reshape(1, H2), Wfc, bfc.reshape(1, 1))
    return outp[:N, 0]


# speedup vs baseline: 1.7264x; 1.0327x over previous
import functools

import jax
import jax.numpy as jnp
from jax import lax
from jax.experimental import pallas as pl
from jax.experimental.pallas import tpu as pltpu
from jax.experimental.pallas import tpu_sc as plsc

N = 10000
E = 160000
F_IN = 256
H1 = 64
H2 = 32

NC = 2
NS = 16
NW = NC * NS

NPAD = 10112
RPS = NPAD // NS
K = 128
EW = 5120
CH = EW // K
EPAD = NW * EW

_MESH = dict(core_axis_name="c", subcore_axis_name="s")


DW = 16

@functools.partial(
    pl.kernel,
    out_type=jax.ShapeDtypeStruct((NC, NPAD, DW), jnp.float32),
    mesh=plsc.VectorSubcoreMesh(**_MESH),
    compiler_params=pltpu.CompilerParams(use_tc_tiling_on_sc=False),
    scratch_types=[
        pltpu.VMEM((CH, K), jnp.int32),
        pltpu.VMEM((K, DW), jnp.float32),
        pltpu.VMEM((RPS, DW), jnp.float32),
        pltpu.VMEM_SHARED((NPAD, DW), jnp.float32),
        pltpu.SemaphoreType.DMA,
    ],
)
def _deg_kernel(dst_hbm, out_hbm, dst_v, ones_v, zbuf_v, acc_sh, ssem):
    c = lax.axis_index("c")
    s = lax.axis_index("s")
    wid = c * NS + s
    row0 = s * RPS
    one16 = jnp.ones((16,), jnp.float32)
    zero16 = jnp.zeros((16,), jnp.float32)

    def fill(i, carry):
        ones_v[i] = one16
        return carry

    lax.fori_loop(0, K, fill, 0)

    def zfill(i, carry):
        zbuf_v[i] = zero16
        return carry

    lax.fori_loop(0, RPS, zfill, 0)
    pltpu.sync_copy(zbuf_v, acc_sh.at[pl.ds(row0, RPS)])
    pltpu.sync_copy(dst_hbm.at[wid], dst_v)
    plsc.subcore_barrier()

    def chunk(j, carry):
        pltpu.async_copy(ones_v, acc_sh.at[dst_v.at[j]], ssem, add=True)
        return carry

    lax.fori_loop(0, CH, chunk, 0)

    def drain(j, carry):
        pltpu.make_async_copy(ones_v, acc_sh.at[pl.ds(0, K)], ssem).wait()
        return carry

    lax.fori_loop(0, CH, drain, 0)
    plsc.subcore_barrier()
    pltpu.sync_copy(acc_sh.at[pl.ds(row0, RPS)], out_hbm.at[c, pl.ds(row0, RPS)])


def _make_prop(F):
    @functools.partial(
        pl.kernel,
        out_type=jax.ShapeDtypeStruct((NC, NPAD, F), jnp.float32),
        mesh=plsc.VectorSubcoreMesh(**_MESH),
        compiler_params=pltpu.CompilerParams(use_tc_tiling_on_sc=False),
        scratch_types=[
            pltpu.VMEM((CH, K), jnp.int32),
            pltpu.VMEM((CH, K), jnp.int32),
            pltpu.VMEM((K, F), jnp.float32),
            pltpu.VMEM((K, F), jnp.float32),
            pltpu.VMEM((K, F), jnp.float32),
            pltpu.VMEM((K, F), jnp.float32),
            pltpu.VMEM_SHARED((NPAD, F), jnp.float32),
            pltpu.SemaphoreType.DMA,
            pltpu.SemaphoreType.DMA,
            pltpu.SemaphoreType.DMA,
            pltpu.SemaphoreType.DMA,
            pltpu.SemaphoreType.DMA,
            pltpu.SemaphoreType.DMA,
            pltpu.SemaphoreType.DMA,
            pltpu.SemaphoreType.DMA,
        ],
    )
    def _prop(h_hbm, src_hbm, dst_hbm, out_hbm, src_v, dst_v,
              rows0_v, rows1_v, rows2_v, rows3_v, acc_sh,
              g0, g1, g2, g3, s0, s1, s2, s3):
        c = lax.axis_index("c")
        s = lax.axis_index("s")
        wid = c * NS + s
        row0 = s * RPS
        rows = [rows0_v, rows1_v, rows2_v, rows3_v]
        gsem = [g0, g1, g2, g3]
        ssem = [s0, s1, s2, s3]
        pltpu.sync_copy(h_hbm.at[pl.ds(row0, RPS)], acc_sh.at[pl.ds(row0, RPS)])
        pltpu.sync_copy(src_hbm.at[wid], src_v)
        pltpu.sync_copy(dst_hbm.at[wid], dst_v)
        plsc.subcore_barrier()

        pltpu.async_copy(h_hbm.at[src_v.at[0]], rows[0], gsem[0])
        pltpu.async_copy(h_hbm.at[src_v.at[1]], rows[1], gsem[1])

        def quad(t, carry):
            j = t * 4
            for b in range(4):
                pltpu.make_async_copy(h_hbm.at[pl.ds(0, K)], rows[b], gsem[b]).wait()
                pltpu.async_copy(rows[b], acc_sh.at[dst_v.at[j + b]], ssem[b],
                                 add=True)
                bn = (b + 2) % 4

                @pl.when(j + b + 2 < CH)
                def _():
                    @pl.when(j + b >= 2)
                    def _():
                        pltpu.make_async_copy(
                            rows[bn], acc_sh.at[pl.ds(0, K)], ssem[bn]).wait()

                    pltpu.async_copy(h_hbm.at[src_v.at[j + b + 2]], rows[bn],
                                     gsem[bn])

            return carry

        lax.fori_loop(0, CH // 4, quad, 0)
        for b in range(4):
            pltpu.make_async_copy(rows[b], acc_sh.at[pl.ds(0, K)], ssem[b]).wait()
        plsc.subcore_barrier()
        pltpu.sync_copy(acc_sh.at[pl.ds(row0, RPS)], out_hbm.at[c, pl.ds(row0, RPS)])

    return _prop


_prop64 = _make_prop(H1)
_prop32 = _make_prop(H2)


def _b1_body(x_ref, w1_ref, xw_ref):
    h = jnp.dot(x_ref[...], w1_ref[...], preferred_element_type=jnp.float32)
    xw_ref[0:N, :] = h
    xw_ref[N:NPAD, :] = jnp.zeros((NPAD - N, H1), jnp.float32)


_b1_call = pl.pallas_call(
    _b1_body,
    out_shape=jax.ShapeDtypeStruct((NPAD, H1), jnp.float32),
)


def _b2_body(hist_ref, xw_ref, h1_ref, dinv_ref):
    deg = hist_ref[0, :, 0:1] + hist_ref[1, :, 0:1] + 1.0
    di = lax.rsqrt(jnp.maximum(deg, 1e-12))
    h1_ref[...] = xw_ref[...] * di
    dinv_ref[...] = di


_b2_call = pl.pallas_call(
    _b2_body,
    out_shape=[
        jax.ShapeDtypeStruct((NPAD, H1), jnp.float32),
        jax.ShapeDtypeStruct((NPAD, 1), jnp.float32),
    ],
)


def _d_body(p_ref, h1_ref, dinv_ref, b1_ref, w2_ref, h2_ref):
    di = dinv_ref[...]
    ssum = p_ref[0] + p_ref[1] - h1_ref[...]
    a1 = jnp.maximum(ssum * di + b1_ref[...], 0.0)
    h2_ref[...] = jnp.dot(a1, w2_ref[...], preferred_element_type=jnp.float32) * di


_d_call = pl.pallas_call(
    _d_body,
    out_shape=jax.ShapeDtypeStruct((NPAD, H2), jnp.float32),
)


def _f_body(q_ref, h2_ref, dinv_ref, b2_ref, wfc_ref, bfc_ref, out_ref):
    di = dinv_ref[...]
    ssum = q_ref[0] + q_ref[1] - h2_ref[...]
    a2 = jnp.maximum(ssum * di + b2_ref[...], 0.0)
    out_ref[...] = jnp.dot(a2, wfc_ref[...], preferred_element_type=jnp.float32) + bfc_ref[...]


_f_call = pl.pallas_call(
    _f_body,
    out_shape=jax.ShapeDtypeStruct((NPAD, 1), jnp.float32),
)


def kernel(x, edge_index, W1, b1, W2, b2, Wfc, bfc):
    src = edge_index[0]
    dst = edge_index[1]
    pad = N + jnp.arange(EPAD - E, dtype=jnp.int32) % (NPAD - N)
    srcp = jnp.concatenate([src, pad]).reshape(NW, CH, K)
    dstp = jnp.concatenate([dst, pad]).reshape(NW, CH, K)

    hist = _deg_kernel(dstp)
    xw = _b1_call(x, W1)
    h1p, dinv = _b2_call(hist, xw)
    p = _prop64(h1p, srcp, dstp)
    h2p = _d_call(p, h1p, dinv, b1.reshape(1, H1), W2)
    q = _prop32(h2p, srcp, dstp)
    outp = _f_call(q, h2p, dinv, b2.reshape(1, H2), Wfc, bfc.reshape(1, 1))
    return outp[:N, 0]
